# baseline (speedup 1.0000x reference)
import functools

import jax
import jax.numpy as jnp
from jax import lax
from jax.experimental import pallas as pl
from jax.experimental.pallas import tpu as pltpu
from jax.experimental.pallas import tpu_sc as plsc


_SC_CORES = 2
_SC_SUBCORES = 16
_LANES = 16
_ROWS_PER_STEP = 4
_RING = 4

_BLK_L = 512

_SC_FRAC_NUM, _SC_FRAC_DEN = 1, 1


def _tc_add_kernel(x_ref, emb_ref, out_ref):
    out_ref[...] = x_ref[...] + emb_ref[...][None, :, :]


def _tc_add(x, emb, l_len):
    b, _, d = x.shape
    blk_l = _BLK_L if l_len % _BLK_L == 0 else l_len
    grid = (l_len // blk_l,)
    return pl.pallas_call(
        _tc_add_kernel,
        grid=grid,
        in_specs=[
            pl.BlockSpec((b, blk_l, d), lambda l: (0, l, 0)),
            pl.BlockSpec((blk_l, d), lambda l: (l, 0)),
        ],
        out_specs=pl.BlockSpec((b, blk_l, d), lambda l: (0, l, 0)),
        out_shape=jax.ShapeDtypeStruct((b, l_len, d), x.dtype),
    )(x, emb)


def _sc_add(x, emb, l_off, l_len):
    b, _, d = x.shape
    nw = _SC_CORES * _SC_SUBCORES
    rows_w = l_len // nw
    r = _ROWS_PER_STEP
    steps = rows_w // r
    vecs = d // _LANES
    mesh = plsc.VectorSubcoreMesh(core_axis_name="c", subcore_axis_name="s")

    @functools.partial(
        pl.kernel,
        out_type=jax.ShapeDtypeStruct((b, l_len, d), jnp.float32),
        mesh=mesh,
        scratch_types=[
            pltpu.VMEM((_RING, b, r, d), jnp.float32),
            pltpu.VMEM((_RING, r, d), jnp.float32),
            [pltpu.SemaphoreType.DMA] * _RING,
            [pltpu.SemaphoreType.DMA] * _RING,
        ],
    )
    def k(x_hbm, emb_hbm, out_hbm, xb, eb, sin, sout):
        wid = lax.axis_index("s") * _SC_CORES + lax.axis_index("c")
        base = wid * rows_w

        def fire_in(j, u):
            l0 = base + j * r
            pltpu.async_copy(
                x_hbm.at[:, pl.ds(l_off + l0, r), :], xb.at[u], sin[u]
            )
            pltpu.async_copy(
                emb_hbm.at[pl.ds(l_off + l0, r), :], eb.at[u], sin[u]
            )

        def wait_in(u):
            pltpu.make_async_copy(
                x_hbm.at[:, pl.ds(l_off, r), :], xb.at[u], sin[u]
            ).wait()
            pltpu.make_async_copy(
                emb_hbm.at[pl.ds(l_off, r), :], eb.at[u], sin[u]
            ).wait()

        def fire_out(j, u):
            l0 = base + j * r
            pltpu.async_copy(xb.at[u], out_hbm.at[:, pl.ds(l0, r), :], sout[u])

        def wait_out(u):
            pltpu.make_async_copy(
                xb.at[u], out_hbm.at[:, pl.ds(base, r), :], sout[u]
            ).wait()

        def compute(u):
            @plsc.parallel_loop(0, r * vecs, 1, unroll=4)
            def _(j):
                row = j // vecs
                col = (j % vecs) * _LANES
                e = eb[u, row, pl.ds(col, _LANES)]
                for bi in range(b):
                    xb[u, bi, row, pl.ds(col, _LANES)] += e

        fire_in(0, 0)
        if steps > 1:
            fire_in(1, 1)
        for i in range(steps):
            u = i % _RING
            j = i + 2
            if j < steps:
                uj = j % _RING
                if j >= _RING:
                    wait_out(uj)
                fire_in(j, uj)
            wait_in(u)
            compute(u)
            fire_out(i, u)
        for i in range(max(steps - _RING, 0), steps):
            wait_out(i % _RING)

    return k(x, emb)


def kernel(x, emb):
    b, seq_len, d = x.shape
    max_len = emb.shape[0]
    if seq_len > max_len:
        x = x[:, :max_len, :]
        seq_len = max_len
    nw = _SC_CORES * _SC_SUBCORES
    rows_unit = nw * _ROWS_PER_STEP
    sc_len = (seq_len * _SC_FRAC_NUM // _SC_FRAC_DEN) // rows_unit * rows_unit
    tc_len = seq_len - sc_len
    sc_ok = (
        x.dtype == jnp.float32
        and emb.dtype == jnp.float32
        and d % _LANES == 0
        and sc_len > 0
        and (tc_len == 0 or tc_len % _BLK_L == 0)
        and _RING * (b + 1) * _ROWS_PER_STEP * d * 4 < 500_000
    )
    if not sc_ok:
        return _tc_add(x, emb, seq_len)
    if tc_len == 0:
        return _sc_add(x, emb, 0, seq_len)
    tc_out = _tc_add(x, emb, tc_len)
    sc_out = _sc_add(x, emb, tc_len, sc_len)
    return jnp.concatenate([tc_out, sc_out], axis=1)


# speedup vs baseline: 1.0701x; 1.0701x over previous
import functools

import jax
import jax.numpy as jnp
from jax import lax
from jax.experimental import pallas as pl
from jax.experimental.pallas import tpu as pltpu
from jax.experimental.pallas import tpu_sc as plsc


_SC_CORES = 2
_SC_SUBCORES = 16
_LANES = 16
_ROWS_PER_STEP = 4
_RING = 4

_BLK_L = 512

_SC_FRAC_NUM, _SC_FRAC_DEN = 1, 1


def _tc_add_kernel(x_ref, emb_ref, out_ref):
    out_ref[...] = x_ref[...] + emb_ref[...][None, :, :]


def _tc_add(x, emb, l_len):
    b, _, d = x.shape
    blk_l = _BLK_L if l_len % _BLK_L == 0 else l_len
    grid = (l_len // blk_l,)
    return pl.pallas_call(
        _tc_add_kernel,
        grid=grid,
        in_specs=[
            pl.BlockSpec((b, blk_l, d), lambda l: (0, l, 0)),
            pl.BlockSpec((blk_l, d), lambda l: (l, 0)),
        ],
        out_specs=pl.BlockSpec((b, blk_l, d), lambda l: (0, l, 0)),
        out_shape=jax.ShapeDtypeStruct((b, l_len, d), x.dtype),
    )(x, emb)


def _sc_add(x, emb, l_off, l_len):
    b, _, d = x.shape
    nw = _SC_CORES * _SC_SUBCORES
    rows_w = l_len // nw
    r = _ROWS_PER_STEP
    steps = rows_w // r
    vecs = d // _LANES
    mesh = plsc.VectorSubcoreMesh(core_axis_name="c", subcore_axis_name="s")

    @functools.partial(
        pl.kernel,
        out_type=jax.ShapeDtypeStruct((b, l_len, d), jnp.float32),
        mesh=mesh,
        scratch_types=[
            pltpu.VMEM((_RING, b, r, d), jnp.float32),
            pltpu.VMEM((_RING, r, d), jnp.float32),
            [pltpu.SemaphoreType.DMA] * _RING,
            [pltpu.SemaphoreType.DMA] * _RING,
        ],
    )
    def k(x_hbm, emb_hbm, out_hbm, xb, eb, sin, sout):
        wid = lax.axis_index("s") * _SC_CORES + lax.axis_index("c")
        base = wid * rows_w

        def fire_in(j, u):
            l0 = base + j * r
            pltpu.async_copy(
                x_hbm.at[:, pl.ds(l_off + l0, r), :], xb.at[u], sin[u]
            )
            pltpu.async_copy(
                emb_hbm.at[pl.ds(l_off + l0, r), :], eb.at[u], sin[u]
            )

        def wait_in(u):
            pltpu.make_async_copy(
                x_hbm.at[:, pl.ds(l_off, r), :], xb.at[u], sin[u]
            ).wait()
            pltpu.make_async_copy(
                emb_hbm.at[pl.ds(l_off, r), :], eb.at[u], sin[u]
            ).wait()

        def fire_out(j, u):
            l0 = base + j * r
            pltpu.async_copy(xb.at[u], out_hbm.at[:, pl.ds(l0, r), :], sout[u])

        def wait_out(u):
            pltpu.make_async_copy(
                xb.at[u], out_hbm.at[:, pl.ds(base, r), :], sout[u]
            ).wait()

        def compute(u):
            @plsc.parallel_loop(0, r * vecs, 1, unroll=4)
            def _(j):
                row = j // vecs
                col = (j % vecs) * _LANES
                e = eb[u, row, pl.ds(col, _LANES)]
                for bi in range(b):
                    xb[u, bi, row, pl.ds(col, _LANES)] += e

        fire_in(0, 0)
        fire_in(1, 1)

        def body(ii, carry):
            for u in range(_RING):
                i = ii * _RING + u
                j = i + 2
                uj = (u + 2) % _RING

                @pl.when(j < steps)
                def _():
                    @pl.when(i >= 2)
                    def _():
                        wait_out(uj)

                    fire_in(j, uj)

                wait_in(u)
                compute(u)
                fire_out(i, u)
            return carry

        lax.fori_loop(0, steps // _RING, body, 0)
        for u in range(_RING):
            wait_out(u)

    return k(x, emb)


def kernel(x, emb):
    b, seq_len, d = x.shape
    max_len = emb.shape[0]
    if seq_len > max_len:
        x = x[:, :max_len, :]
        seq_len = max_len
    nw = _SC_CORES * _SC_SUBCORES
    rows_unit = nw * _ROWS_PER_STEP * _RING
    sc_len = (seq_len * _SC_FRAC_NUM // _SC_FRAC_DEN) // rows_unit * rows_unit
    tc_len = seq_len - sc_len
    sc_ok = (
        x.dtype == jnp.float32
        and emb.dtype == jnp.float32
        and d % _LANES == 0
        and sc_len > 0
        and (tc_len == 0 or tc_len % _BLK_L == 0)
        and _RING * (b + 1) * _ROWS_PER_STEP * d * 4 < 500_000
    )
    if not sc_ok:
        return _tc_add(x, emb, seq_len)
    if tc_len == 0:
        return _sc_add(x, emb, 0, seq_len)
    tc_out = _tc_add(x, emb, tc_len)
    sc_out = _sc_add(x, emb, tc_len, sc_len)
    return jnp.concatenate([tc_out, sc_out], axis=1)
